# Initial kernel scaffold; baseline (speedup 1.0000x reference)
import jax
import jax.numpy as jnp
from jax.experimental import pallas as pl


def kernel(x, edge_index, W1, b1, W2, b2):
    raise NotImplementedError("write your pallas kernel here")


# speedup vs baseline: 8.8752x; 8.8752x over previous
import functools

import jax
import jax.numpy as jnp
from jax import lax
from jax.experimental import pallas as pl
from jax.experimental.pallas import tpu as pltpu
from jax.experimental.pallas import tpu_sc as plsc

NC = 2
NS = 16
NW = NC * NS
CH = 128


def _zero_rows(zbuf, nrows, width):
  z16 = jnp.zeros((16,), jnp.float32)

  def body(i, c):
    for j in range(width // 16):
      zbuf[i, pl.ds(j * 16, 16)] = z16
    return c

  lax.fori_loop(0, nrows, body, 0)


def _make_deg_kernel(npad, rpt, ept, nchunks):
  mesh = plsc.VectorSubcoreMesh(core_axis_name="c", subcore_axis_name="s")
  zr = rpt // 4

  def body(dst_hbm, out_hbm, idx_v, ones_v, zbuf, deg_sh):
    cid = lax.axis_index("c")
    sid = lax.axis_index("s")
    base = (cid * NS + sid) * ept

    o16 = jnp.ones((16,), jnp.float32)

    def fill(i, c):
      ones_v[i, :] = o16
      return c

    lax.fori_loop(0, CH, fill, 0)
    _zero_rows(zbuf, zr, 16)

    def zcopy(i, c):
      pltpu.sync_copy(zbuf, deg_sh.at[pl.ds(sid * rpt + i * zr, zr)])
      return c

    lax.fori_loop(0, rpt // zr, zcopy, 0)
    plsc.subcore_barrier()

    def step(i, c):
      pltpu.sync_copy(dst_hbm.at[pl.ds(base + i * CH, CH)], idx_v)
      pltpu.sync_copy(ones_v, deg_sh.at[idx_v], add=True)
      return c

    lax.fori_loop(0, nchunks, step, 0)
    plsc.subcore_barrier()
    pltpu.sync_copy(deg_sh.at[pl.ds(sid * rpt, rpt)],
                    out_hbm.at[cid, pl.ds(sid * rpt, rpt), :])

  return pl.kernel(
      body,
      out_type=jax.ShapeDtypeStruct((NC, npad, 16), jnp.float32),
      mesh=mesh,
      compiler_params=pltpu.CompilerParams(use_tc_tiling_on_sc=False),
      scratch_types=[
          pltpu.VMEM((CH,), jnp.int32),
          pltpu.VMEM((CH, 16), jnp.float32),
          pltpu.VMEM((zr, 16), jnp.float32),
          pltpu.VMEM_SHARED((npad, 16), jnp.float32),
      ],
  )


def _make_agg_kernel(npad, rpt, ept, nchunks, d):
  mesh = plsc.VectorSubcoreMesh(core_axis_name="c", subcore_axis_name="s")
  zr = rpt // 4

  def body(g_hbm, src_hbm, dst_hbm, out_hbm, sidx, didx, rows, zbuf, accum,
           sem):
    cid = lax.axis_index("c")
    sid = lax.axis_index("s")
    base = (cid * NS + sid) * ept

    _zero_rows(zbuf, zr, d)

    def zcopy(i, c):
      pltpu.sync_copy(zbuf, accum.at[pl.ds(sid * rpt + i * zr, zr)])
      return c

    lax.fori_loop(0, rpt // zr, zcopy, 0)
    plsc.subcore_barrier()

    def step(i, c):
      pltpu.sync_copy(src_hbm.at[pl.ds(base + i * CH, CH)], sidx)
      pltpu.sync_copy(dst_hbm.at[pl.ds(base + i * CH, CH)], didx)
      pltpu.async_copy(g_hbm.at[sidx], rows, sem).wait()
      pltpu.sync_copy(rows, accum.at[didx], add=True)
      return c

    lax.fori_loop(0, nchunks, step, 0)
    plsc.subcore_barrier()
    pltpu.sync_copy(accum.at[pl.ds(sid * rpt, rpt)],
                    out_hbm.at[cid, pl.ds(sid * rpt, rpt), :])

  return pl.kernel(
      body,
      out_type=jax.ShapeDtypeStruct((NC, npad, d), jnp.float32),
      mesh=mesh,
      compiler_params=pltpu.CompilerParams(use_tc_tiling_on_sc=False),
      scratch_types=[
          pltpu.VMEM((CH,), jnp.int32),
          pltpu.VMEM((CH,), jnp.int32),
          pltpu.VMEM((CH, d), jnp.float32),
          pltpu.VMEM((zr, d), jnp.float32),
          pltpu.VMEM_SHARED((npad, d), jnp.float32),
          pltpu.SemaphoreType.DMA,
      ],
  )


def _prep1(x_p, d0, d1, w1, npad, br, k, hid):
  def body(x_ref, d0_ref, d1_ref, w_ref, g_ref, dinv_ref):
    deg = d0_ref[...] + d1_ref[...] + 1.0
    dinv = lax.rsqrt(deg)
    dinv_ref[...] = dinv
    xs = x_ref[...] * dinv[:, 0:1]
    g_ref[...] = lax.dot_general(xs, w_ref[...], (((1,), (1,)), ((), ())),
                                 preferred_element_type=jnp.float32)

  return pl.pallas_call(
      body,
      grid=(npad // br,),
      in_specs=[
          pl.BlockSpec((br, k), lambda i: (i, 0)),
          pl.BlockSpec((br, 16), lambda i: (i, 0)),
          pl.BlockSpec((br, 16), lambda i: (i, 0)),
          pl.BlockSpec((hid, k), lambda i: (0, 0)),
      ],
      out_specs=[
          pl.BlockSpec((br, hid), lambda i: (i, 0)),
          pl.BlockSpec((br, 16), lambda i: (i, 0)),
      ],
      out_shape=[
          jax.ShapeDtypeStruct((npad, hid), jnp.float32),
          jax.ShapeDtypeStruct((npad, 16), jnp.float32),
      ],
  )(x_p, d0, d1, w1)


def _prep2(s0, s1, g1, dinv, b1, w2, npad, br, hid, lat):
  def body(s0_ref, s1_ref, g1_ref, dinv_ref, b_ref, w_ref, out_ref):
    dv = dinv_ref[...][:, 0:1]
    z = dv * (s0_ref[...] + s1_ref[...] + g1_ref[...]) + b_ref[...]
    z = jnp.maximum(z, 0.0) * dv
    out_ref[...] = lax.dot_general(z, w_ref[...], (((1,), (1,)), ((), ())),
                                   preferred_element_type=jnp.float32)

  return pl.pallas_call(
      body,
      grid=(npad // br,),
      in_specs=[
          pl.BlockSpec((br, hid), lambda i: (i, 0)),
          pl.BlockSpec((br, hid), lambda i: (i, 0)),
          pl.BlockSpec((br, hid), lambda i: (i, 0)),
          pl.BlockSpec((br, 16), lambda i: (i, 0)),
          pl.BlockSpec((1, hid), lambda i: (0, 0)),
          pl.BlockSpec((lat, hid), lambda i: (0, 0)),
      ],
      out_specs=pl.BlockSpec((br, lat), lambda i: (i, 0)),
      out_shape=jax.ShapeDtypeStruct((npad, lat), jnp.float32),
  )(s0, s1, g1, dinv, b1, w2)


def _finalize(s0, s1, g2, dinv, b2, npad, br, lat):
  def body(s0_ref, s1_ref, g2_ref, dinv_ref, b_ref, out_ref):
    dv = dinv_ref[...][:, 0:1]
    out_ref[...] = dv * (s0_ref[...] + s1_ref[...] + g2_ref[...]) + b_ref[...]

  return pl.pallas_call(
      body,
      grid=(npad // br,),
      in_specs=[
          pl.BlockSpec((br, lat), lambda i: (i, 0)),
          pl.BlockSpec((br, lat), lambda i: (i, 0)),
          pl.BlockSpec((br, lat), lambda i: (i, 0)),
          pl.BlockSpec((br, 16), lambda i: (i, 0)),
          pl.BlockSpec((1, lat), lambda i: (0, 0)),
      ],
      out_specs=pl.BlockSpec((br, lat), lambda i: (i, 0)),
      out_shape=jax.ShapeDtypeStruct((npad, lat), jnp.float32),
  )(s0, s1, g2, dinv, b2)


def kernel(x, edge_index, W1, b1, W2, b2):
  n, k = x.shape
  hid = W1.shape[0]
  lat = W2.shape[0]

  src = edge_index[0].astype(jnp.int32)
  dst = edge_index[1].astype(jnp.int32)
  e = src.shape[0]

  ept = -(-e // (NW * 2 * CH)) * (2 * CH)
  epad = ept * NW
  nchunks = ept // CH
  src_p = jnp.concatenate([src, jnp.full((epad - e,), n, jnp.int32)])
  dst_p = jnp.concatenate([dst, jnp.full((epad - e,), n, jnp.int32)])

  rpt = -(-(n + 1) // NS)
  rpt = -(-rpt // 8) * 8
  npad = rpt * NS
  br = rpt

  x_p = jnp.pad(x, ((0, npad - n), (0, 0)))

  deg = _make_deg_kernel(npad, rpt, ept, nchunks)(dst_p)
  g1, dinv = _prep1(x_p, deg[0], deg[1], W1, npad, br, k, hid)
  s1 = _make_agg_kernel(npad, rpt, ept, nchunks, hid)(g1, src_p, dst_p)
  g2 = _prep2(s1[0], s1[1], g1, dinv, b1.reshape(1, hid), W2, npad, br, hid,
              lat)
  s2 = _make_agg_kernel(npad, rpt, ept, nchunks, lat)(g2, src_p, dst_p)
  out = _finalize(s2[0], s2[1], g2, dinv, b2.reshape(1, lat), npad, br, lat)
  return out[:n]
